# baseline (speedup 1.0000x reference)
import jax
import jax.numpy as jnp
from jax import lax
from jax.experimental import pallas as pl
from jax.experimental.pallas import tpu as pltpu
from jax.experimental.pallas import tpu_sc as plsc

N = 10000
E = 320000
D_IN = 128
D_OUT = 10
DP = 16
NC = 2
NS = 16
NW = NC * NS
E_T = E // NW
CH = 80
MAIN = 125
PAIRS = (MAIN - 1) // 2
RT = 624
RT_LAST = N - RT * (NS - 1)
EPS = 1e-8

_mesh = plsc.VectorSubcoreMesh(core_axis_name="c", subcore_axis_name="s")


def _sc_agg1_body(h_hbm, src_hbm, dst_hbm, z128_hbm, z16_hbm, ones_hbm,
                  out_agg, out_deg, acc, dega, dst_v, s0, s1, s2,
                  g0, g1, g2, ones_v, ig0, ig1, ig2, sg0, sg1, sg2,
                  ss0, ss1, ss2, degsem):
    cid = lax.axis_index("c")
    sid = lax.axis_index("s")
    wid = cid * NS + sid
    rbase = sid * RT
    sv = [s0, s1, s2]
    gv = [g0, g1, g2]
    igs = [ig0, ig1, ig2]
    sgs = [sg0, sg1, sg2]
    sss = [ss0, ss1, ss2]

    def zero(sz):
        def f():
            pltpu.sync_copy(z128_hbm.at[pl.ds(0, sz)],
                            acc.at[pl.ds(rbase, sz)])
            pltpu.sync_copy(z16_hbm.at[pl.ds(0, sz)],
                            dega.at[pl.ds(rbase, sz)])
        return f

    pl.when(sid < NS - 1)(zero(RT))
    pl.when(sid == NS - 1)(zero(RT_LAST))
    pltpu.sync_copy(ones_hbm, ones_v)
    pltpu.sync_copy(dst_hbm.at[wid], dst_v)
    for b in range(3):
        pltpu.sync_copy(src_hbm.at[wid, b], sv[b])
    plsc.subcore_barrier()

    def launch(b):
        pltpu.async_copy(h_hbm.at[sv[b]], gv[b], sgs[b])

    def draing(b):
        pltpu.make_async_copy(h_hbm.at[pl.ds(0, CH)], gv[b], sgs[b]).wait()

    def drains(b):
        pltpu.make_async_copy(h_hbm.at[pl.ds(0, CH)], gv[b], sss[b]).wait()

    launch(0)
    launch(1)

    def step(c, b):
        nb = (b + 2) % 3
        draing(b)

        def spre():
            pltpu.async_copy(src_hbm.at[wid, c + 3], sv[b], igs[b])

        pl.when(c + 3 < MAIN)(spre)
        pltpu.async_copy(gv[b], acc.at[dst_v.at[c]], sss[b], add=True)
        pltpu.async_copy(ones_v, dega.at[dst_v.at[c]], degsem, add=True)
        pl.when(c >= 1)(lambda: drains(nb))

        def lg_warm():
            pltpu.make_async_copy(src_hbm.at[0, 0], sv[nb], igs[nb]).wait()
            launch(nb)

        def lg_cold():
            launch(nb)

        pl.when((c + 2 < MAIN) & (c >= 1))(lg_warm)
        pl.when((c + 2 < MAIN) & (c < 1))(lg_cold)
        return

    def body(k, carry):
        for j in range(3):
            step(3 * k + j, j)
        return carry

    lax.fori_loop(0, MAIN // 3, body, 0)
    step(MAIN - 2, (MAIN - 2) % 3)
    step(MAIN - 1, (MAIN - 1) % 3)
    drains((MAIN - 1) % 3)

    def degdrain(i, carry):
        pltpu.make_async_copy(ones_hbm, ones_v, degsem).wait()
        return carry

    lax.fori_loop(0, MAIN, degdrain, 0)
    plsc.subcore_barrier()

    def wb(sz):
        def f():
            pltpu.sync_copy(acc.at[pl.ds(rbase, sz)],
                            out_agg.at[cid, pl.ds(rbase, sz)])
            pltpu.sync_copy(dega.at[pl.ds(rbase, sz)],
                            out_deg.at[cid, pl.ds(rbase, sz)])
        return f

    pl.when(sid < NS - 1)(wb(RT))
    pl.when(sid == NS - 1)(wb(RT_LAST))


_sc_agg1 = pl.kernel(
    _sc_agg1_body,
    out_type=(jax.ShapeDtypeStruct((NC, N, D_IN), jnp.float32),
              jax.ShapeDtypeStruct((NC, N, DP), jnp.bfloat16)),
    mesh=_mesh,
    name="sc_agg1",
    compiler_params=pltpu.CompilerParams(use_tc_tiling_on_sc=False),
    scratch_types=[
        pltpu.VMEM_SHARED((N, D_IN), jnp.float32),
        pltpu.VMEM_SHARED((N, DP), jnp.bfloat16),
        pltpu.VMEM((MAIN, CH), jnp.int32),
        pltpu.VMEM((CH,), jnp.int32),
        pltpu.VMEM((CH,), jnp.int32),
        pltpu.VMEM((CH,), jnp.int32),
        pltpu.VMEM((CH, D_IN), jnp.float32),
        pltpu.VMEM((CH, D_IN), jnp.float32),
        pltpu.VMEM((CH, D_IN), jnp.float32),
        pltpu.VMEM((CH, DP), jnp.bfloat16),
        pltpu.SemaphoreType.DMA,
        pltpu.SemaphoreType.DMA,
        pltpu.SemaphoreType.DMA,
        pltpu.SemaphoreType.DMA,
        pltpu.SemaphoreType.DMA,
        pltpu.SemaphoreType.DMA,
        pltpu.SemaphoreType.DMA,
        pltpu.SemaphoreType.DMA,
        pltpu.SemaphoreType.DMA,
        pltpu.SemaphoreType.DMA,
    ],
)


RING = 5
QUINTS = MAIN // RING
LOOK = 4


def _sc_agg2_body(p_hbm, src_hbm, dst_hbm, z16_hbm, out_agg, acc,
                  src_v, dst_v, b0, b1, b2, b3, b4,
                  sg0, sg1, sg2, sg3, sg4, ss0, ss1, ss2, ss3, ss4):
    cid = lax.axis_index("c")
    sid = lax.axis_index("s")
    wid = cid * NS + sid
    rbase = sid * RT
    bufs = [b0, b1, b2, b3, b4]
    gsems = [sg0, sg1, sg2, sg3, sg4]
    ssems = [ss0, ss1, ss2, ss3, ss4]

    def zero(sz):
        def f():
            pltpu.sync_copy(z16_hbm.at[pl.ds(0, sz)],
                            acc.at[pl.ds(rbase, sz)])
        return f

    pl.when(sid < NS - 1)(zero(RT))
    pl.when(sid == NS - 1)(zero(RT_LAST))
    pltpu.sync_copy(src_hbm.at[wid], src_v)
    pltpu.sync_copy(dst_hbm.at[wid], dst_v)
    plsc.subcore_barrier()

    def launch(i, b):
        pltpu.async_copy(p_hbm.at[src_v.at[i]], bufs[b], gsems[b])

    def draing(b):
        pltpu.make_async_copy(p_hbm.at[pl.ds(0, CH)], bufs[b],
                              gsems[b]).wait()

    def drains(b):
        pltpu.make_async_copy(p_hbm.at[pl.ds(0, CH)], bufs[b],
                              ssems[b]).wait()

    launch(0, 0)
    launch(1, 1)
    launch(2, 2)
    launch(3, 3)

    def body(k, carry):
        c0 = RING * k
        for j in range(RING):
            c = c0 + j
            draing(j)
            pltpu.async_copy(bufs[j], acc.at[dst_v.at[c]], ssems[j],
                             add=True)
            nb = (j + LOOK) % RING
            cn = c + LOOK
            pl.when(cn - RING >= 0)(lambda b=nb: drains(b))
            pl.when(cn < MAIN)(lambda i=cn, b=nb: launch(i, b))
        return carry

    lax.fori_loop(0, QUINTS, body, 0)
    drains(4)
    plsc.subcore_barrier()

    def wb(sz):
        def f():
            pltpu.sync_copy(acc.at[pl.ds(rbase, sz)],
                            out_agg.at[cid, pl.ds(rbase, sz)])
        return f

    pl.when(sid < NS - 1)(wb(RT))
    pl.when(sid == NS - 1)(wb(RT_LAST))


_sc_agg2 = pl.kernel(
    _sc_agg2_body,
    out_type=jax.ShapeDtypeStruct((NC, N, DP), jnp.float32),
    mesh=_mesh,
    name="sc_agg2",
    compiler_params=pltpu.CompilerParams(use_tc_tiling_on_sc=False),
    scratch_types=[
        pltpu.VMEM_SHARED((N, DP), jnp.float32),
        pltpu.VMEM((MAIN, CH), jnp.int32),
        pltpu.VMEM((MAIN, CH), jnp.int32),
        pltpu.VMEM((CH, DP), jnp.float32),
        pltpu.VMEM((CH, DP), jnp.float32),
        pltpu.VMEM((CH, DP), jnp.float32),
        pltpu.VMEM((CH, DP), jnp.float32),
        pltpu.VMEM((CH, DP), jnp.float32),
        pltpu.SemaphoreType.DMA,
        pltpu.SemaphoreType.DMA,
        pltpu.SemaphoreType.DMA,
        pltpu.SemaphoreType.DMA,
        pltpu.SemaphoreType.DMA,
        pltpu.SemaphoreType.DMA,
        pltpu.SemaphoreType.DMA,
        pltpu.SemaphoreType.DMA,
        pltpu.SemaphoreType.DMA,
        pltpu.SemaphoreType.DMA,
    ],
)


BLK = 2000


def _t1_body(x_ref, o_ref):
    x = x_ref[...]
    n = jnp.sqrt(jnp.sum(x * x, axis=1, keepdims=True))
    o_ref[...] = x / (n + EPS)


_t1 = pl.pallas_call(
    _t1_body,
    grid=(N // BLK,),
    in_specs=[pl.BlockSpec((BLK, D_IN), lambda i: (i, 0))],
    out_specs=pl.BlockSpec((BLK, D_IN), lambda i: (i, 0)),
    out_shape=jax.ShapeDtypeStruct((N, D_IN), jnp.float32),
)


def _t2_body(h_ref, a0_ref, a1_ref, d0_ref, d1_ref, ws1_ref, wn1_ref,
             b1_ref, ws2_ref, wn2_ref, s_ref, p_ref):
    h = h_ref[...]
    deg = (d0_ref[0].astype(jnp.float32)
           + d1_ref[0].astype(jnp.float32))[:, :1]
    agg = (a0_ref[0] + a1_ref[0]) / jnp.maximum(deg, 1.0)
    z = (jnp.dot(h, ws1_ref[...], preferred_element_type=jnp.float32)
         + jnp.dot(agg, wn1_ref[...], preferred_element_type=jnp.float32)
         + b1_ref[...])
    n = jnp.sqrt(jnp.sum(z * z, axis=1, keepdims=True))
    z = jnp.maximum(z / (n + EPS), 0.0)
    s_ref[...] = jnp.dot(z, ws2_ref[...], preferred_element_type=jnp.float32)
    p_ref[...] = jnp.dot(z, wn2_ref[...], preferred_element_type=jnp.float32)


_t2 = pl.pallas_call(
    _t2_body,
    grid=(N // BLK,),
    in_specs=[
        pl.BlockSpec((BLK, D_IN), lambda i: (i, 0)),
        pl.BlockSpec((1, BLK, D_IN), lambda i: (0, i, 0)),
        pl.BlockSpec((1, BLK, D_IN), lambda i: (1, i, 0)),
        pl.BlockSpec((1, BLK, DP), lambda i: (0, i, 0)),
        pl.BlockSpec((1, BLK, DP), lambda i: (1, i, 0)),
        pl.BlockSpec((D_IN, D_IN), lambda i: (0, 0)),
        pl.BlockSpec((D_IN, D_IN), lambda i: (0, 0)),
        pl.BlockSpec((1, D_IN), lambda i: (0, 0)),
        pl.BlockSpec((D_IN, DP), lambda i: (0, 0)),
        pl.BlockSpec((D_IN, DP), lambda i: (0, 0)),
    ],
    out_specs=[
        pl.BlockSpec((BLK, DP), lambda i: (i, 0)),
        pl.BlockSpec((BLK, DP), lambda i: (i, 0)),
    ],
    out_shape=[
        jax.ShapeDtypeStruct((N, DP), jnp.float32),
        jax.ShapeDtypeStruct((N, DP), jnp.float32),
    ],
)


def _t3_body(s_ref, q0_ref, q1_ref, d0_ref, d1_ref, b2_ref, o_ref):
    deg = (d0_ref[0].astype(jnp.float32)
           + d1_ref[0].astype(jnp.float32))[:, :1]
    agg = (q0_ref[0] + q1_ref[0]) / jnp.maximum(deg, 1.0)
    o = s_ref[...] + agg + b2_ref[...]
    n = jnp.sqrt(jnp.sum(o * o, axis=1, keepdims=True))
    o = o / (n + EPS)
    mask = lax.broadcasted_iota(jnp.int32, o.shape, 1) < D_OUT
    o = jnp.where(mask, o, -jnp.inf)
    m = jnp.max(o, axis=1, keepdims=True)
    e = jnp.where(mask, jnp.exp(o - m), 0.0)
    lse = jnp.log(jnp.sum(e, axis=1, keepdims=True))
    o_ref[...] = (o - m - lse)[:, :D_OUT]


_t3 = pl.pallas_call(
    _t3_body,
    grid=(N // BLK,),
    in_specs=[
        pl.BlockSpec((BLK, DP), lambda i: (i, 0)),
        pl.BlockSpec((1, BLK, DP), lambda i: (0, i, 0)),
        pl.BlockSpec((1, BLK, DP), lambda i: (1, i, 0)),
        pl.BlockSpec((1, BLK, DP), lambda i: (0, i, 0)),
        pl.BlockSpec((1, BLK, DP), lambda i: (1, i, 0)),
        pl.BlockSpec((1, DP), lambda i: (0, 0)),
    ],
    out_specs=pl.BlockSpec((BLK, D_OUT), lambda i: (i, 0)),
    out_shape=jax.ShapeDtypeStruct((N, D_OUT), jnp.float32),
)


@jax.jit
def kernel(x, edge_index, W_self1, W_neigh1, b1, W_self2, W_neigh2, b2):
    src3 = edge_index[0].reshape(NW, MAIN, CH)
    dst3 = edge_index[1].reshape(NW, MAIN, CH)
    z128 = jnp.zeros((RT_LAST, D_IN), jnp.float32)
    z16 = jnp.zeros((RT_LAST, DP), jnp.float32)
    z16b = jnp.zeros((RT_LAST, DP), jnp.bfloat16)
    ones16 = jnp.ones((CH, DP), jnp.bfloat16)
    ws2p = jnp.pad(W_self2, ((0, 0), (0, DP - D_OUT)))
    wn2p = jnp.pad(W_neigh2, ((0, 0), (0, DP - D_OUT)))
    b1r = b1.reshape(1, D_IN)
    b2p = jnp.pad(b2, (0, DP - D_OUT)).reshape(1, DP)

    h = _t1(x)
    agg_parts, deg_parts = _sc_agg1(h, src3, dst3, z128, z16b, ones16)
    s, p = _t2(h, agg_parts, agg_parts, deg_parts, deg_parts,
               W_self1, W_neigh1, b1r, ws2p, wn2p)
    parts2 = _sc_agg2(p, src3, dst3, z16)
    return _t3(s, parts2, parts2, deg_parts, deg_parts, b2p)


# speedup vs baseline: 1.0015x; 1.0015x over previous
import jax
import jax.numpy as jnp
from jax import lax
from jax.experimental import pallas as pl
from jax.experimental.pallas import tpu as pltpu
from jax.experimental.pallas import tpu_sc as plsc

N = 10000
E = 320000
D_IN = 128
D_OUT = 10
DP = 16
NC = 2
NS = 16
NW = NC * NS
CH = 80
MAIN = 125
RT = 624
RT_LAST = N - RT * (NS - 1)
EPS = 1e-8

_mesh = plsc.VectorSubcoreMesh(core_axis_name="c", subcore_axis_name="s")


def _sc_agg1_body(h_hbm, src_hbm, dst_hbm, z128_hbm, z16_hbm, ones_hbm,
                  out_agg, out_deg, acc, dega, dst_v, s0, s1, s2,
                  g0, g1, g2, ones_v, ig0, ig1, ig2, sg0, sg1, sg2,
                  ss0, ss1, ss2, degsem):
    cid = lax.axis_index("c")
    sid = lax.axis_index("s")
    wid = cid * NS + sid
    rbase = sid * RT
    sv = [s0, s1, s2]
    gv = [g0, g1, g2]
    igs = [ig0, ig1, ig2]
    sgs = [sg0, sg1, sg2]
    sss = [ss0, ss1, ss2]

    def zero(sz):
        def f():
            pltpu.sync_copy(z128_hbm.at[pl.ds(0, sz)],
                            acc.at[pl.ds(rbase, sz)])
            pltpu.sync_copy(z16_hbm.at[pl.ds(0, sz)],
                            dega.at[pl.ds(rbase, sz)])
        return f

    pl.when(sid < NS - 1)(zero(RT))
    pl.when(sid == NS - 1)(zero(RT_LAST))
    pltpu.sync_copy(ones_hbm, ones_v)
    pltpu.sync_copy(dst_hbm.at[wid], dst_v)
    for b in range(3):
        pltpu.sync_copy(src_hbm.at[wid, b], sv[b])
    plsc.subcore_barrier()

    def launch(b):
        pltpu.async_copy(h_hbm.at[sv[b]], gv[b], sgs[b])

    def draing(b):
        pltpu.make_async_copy(h_hbm.at[pl.ds(0, CH)], gv[b], sgs[b]).wait()

    def drains(b):
        pltpu.make_async_copy(h_hbm.at[pl.ds(0, CH)], gv[b], sss[b]).wait()

    launch(0)
    launch(1)

    def step(c, b):
        nb = (b + 2) % 3
        draing(b)

        def spre():
            pltpu.async_copy(src_hbm.at[wid, c + 3], sv[b], igs[b])

        pl.when(c + 3 < MAIN)(spre)
        pltpu.async_copy(gv[b], acc.at[dst_v.at[c]], sss[b], add=True)
        pltpu.async_copy(ones_v, dega.at[dst_v.at[c]], degsem, add=True)
        pl.when(c >= 1)(lambda: drains(nb))

        def lg_warm():
            pltpu.make_async_copy(src_hbm.at[0, 0], sv[nb], igs[nb]).wait()
            launch(nb)

        def lg_cold():
            launch(nb)

        pl.when((c + 2 < MAIN) & (c >= 1))(lg_warm)
        pl.when((c + 2 < MAIN) & (c < 1))(lg_cold)
        return

    def body(k, carry):
        for j in range(3):
            step(3 * k + j, j)
        return carry

    lax.fori_loop(0, MAIN // 3, body, 0)
    step(MAIN - 2, (MAIN - 2) % 3)
    step(MAIN - 1, (MAIN - 1) % 3)
    drains((MAIN - 1) % 3)

    def degdrain(i, carry):
        pltpu.make_async_copy(ones_hbm, ones_v, degsem).wait()
        return carry

    lax.fori_loop(0, MAIN, degdrain, 0)
    plsc.subcore_barrier()

    def wb(sz):
        def f():
            pltpu.sync_copy(acc.at[pl.ds(rbase, sz)],
                            out_agg.at[cid, pl.ds(rbase, sz)])
            pltpu.sync_copy(dega.at[pl.ds(rbase, sz)],
                            out_deg.at[cid, pl.ds(rbase, sz)])
        return f

    pl.when(sid < NS - 1)(wb(RT))
    pl.when(sid == NS - 1)(wb(RT_LAST))


_sc_agg1 = pl.kernel(
    _sc_agg1_body,
    out_type=(jax.ShapeDtypeStruct((NC, N, D_IN), jnp.float32),
              jax.ShapeDtypeStruct((NC, N, DP), jnp.bfloat16)),
    mesh=_mesh,
    name="sc_agg1",
    compiler_params=pltpu.CompilerParams(use_tc_tiling_on_sc=False),
    scratch_types=[
        pltpu.VMEM_SHARED((N, D_IN), jnp.float32),
        pltpu.VMEM_SHARED((N, DP), jnp.bfloat16),
        pltpu.VMEM((MAIN, CH), jnp.int32),
        pltpu.VMEM((CH,), jnp.int32),
        pltpu.VMEM((CH,), jnp.int32),
        pltpu.VMEM((CH,), jnp.int32),
        pltpu.VMEM((CH, D_IN), jnp.float32),
        pltpu.VMEM((CH, D_IN), jnp.float32),
        pltpu.VMEM((CH, D_IN), jnp.float32),
        pltpu.VMEM((CH, DP), jnp.bfloat16),
        pltpu.SemaphoreType.DMA,
        pltpu.SemaphoreType.DMA,
        pltpu.SemaphoreType.DMA,
        pltpu.SemaphoreType.DMA,
        pltpu.SemaphoreType.DMA,
        pltpu.SemaphoreType.DMA,
        pltpu.SemaphoreType.DMA,
        pltpu.SemaphoreType.DMA,
        pltpu.SemaphoreType.DMA,
        pltpu.SemaphoreType.DMA,
    ],
)


RING = 5
QUINTS = MAIN // RING
LOOK = 4


def _sc_agg2_body(p_hbm, src_hbm, dst_hbm, z16_hbm, out_agg, acc,
                  src_v, dst_v, b0, b1, b2, b3, b4,
                  sg0, sg1, sg2, sg3, sg4, ss0, ss1, ss2, ss3, ss4):
    cid = lax.axis_index("c")
    sid = lax.axis_index("s")
    wid = cid * NS + sid
    rbase = sid * RT
    bufs = [b0, b1, b2, b3, b4]
    gsems = [sg0, sg1, sg2, sg3, sg4]
    ssems = [ss0, ss1, ss2, ss3, ss4]

    def zero(sz):
        def f():
            pltpu.sync_copy(z16_hbm.at[pl.ds(0, sz)],
                            acc.at[pl.ds(rbase, sz)])
        return f

    pl.when(sid < NS - 1)(zero(RT))
    pl.when(sid == NS - 1)(zero(RT_LAST))
    pltpu.sync_copy(src_hbm.at[wid], src_v)
    pltpu.sync_copy(dst_hbm.at[wid], dst_v)
    plsc.subcore_barrier()

    def launch(i, b):
        pltpu.async_copy(p_hbm.at[src_v.at[i]], bufs[b], gsems[b])

    def draing(b):
        pltpu.make_async_copy(p_hbm.at[pl.ds(0, CH)], bufs[b],
                              gsems[b]).wait()

    def drains(b):
        pltpu.make_async_copy(p_hbm.at[pl.ds(0, CH)], bufs[b],
                              ssems[b]).wait()

    launch(0, 0)
    launch(1, 1)
    launch(2, 2)
    launch(3, 3)

    def body(k, carry):
        c0 = RING * k
        for j in range(RING):
            c = c0 + j
            draing(j)
            pltpu.async_copy(bufs[j], acc.at[dst_v.at[c]], ssems[j],
                             add=True)
            nb = (j + LOOK) % RING
            cn = c + LOOK
            pl.when(cn - RING >= 0)(lambda b=nb: drains(b))
            pl.when(cn < MAIN)(lambda i=cn, b=nb: launch(i, b))
        return carry

    lax.fori_loop(0, QUINTS, body, 0)
    drains(4)
    plsc.subcore_barrier()

    def wb(sz):
        def f():
            pltpu.sync_copy(acc.at[pl.ds(rbase, sz)],
                            out_agg.at[cid, pl.ds(rbase, sz)])
        return f

    pl.when(sid < NS - 1)(wb(RT))
    pl.when(sid == NS - 1)(wb(RT_LAST))


_sc_agg2 = pl.kernel(
    _sc_agg2_body,
    out_type=jax.ShapeDtypeStruct((NC, N, DP), jnp.float32),
    mesh=_mesh,
    name="sc_agg2",
    compiler_params=pltpu.CompilerParams(use_tc_tiling_on_sc=False),
    scratch_types=[
        pltpu.VMEM_SHARED((N, DP), jnp.float32),
        pltpu.VMEM((MAIN, CH), jnp.int32),
        pltpu.VMEM((MAIN, CH), jnp.int32),
        pltpu.VMEM((CH, DP), jnp.float32),
        pltpu.VMEM((CH, DP), jnp.float32),
        pltpu.VMEM((CH, DP), jnp.float32),
        pltpu.VMEM((CH, DP), jnp.float32),
        pltpu.VMEM((CH, DP), jnp.float32),
        pltpu.SemaphoreType.DMA,
        pltpu.SemaphoreType.DMA,
        pltpu.SemaphoreType.DMA,
        pltpu.SemaphoreType.DMA,
        pltpu.SemaphoreType.DMA,
        pltpu.SemaphoreType.DMA,
        pltpu.SemaphoreType.DMA,
        pltpu.SemaphoreType.DMA,
        pltpu.SemaphoreType.DMA,
        pltpu.SemaphoreType.DMA,
    ],
)


BLK = 2000


def _t1_body(x_ref, o_ref):
    x = x_ref[...]
    n = jnp.sqrt(jnp.sum(x * x, axis=1, keepdims=True))
    o_ref[...] = x / (n + EPS)


_t1 = pl.pallas_call(
    _t1_body,
    grid=(N // BLK,),
    in_specs=[pl.BlockSpec((BLK, D_IN), lambda i: (i, 0))],
    out_specs=pl.BlockSpec((BLK, D_IN), lambda i: (i, 0)),
    out_shape=jax.ShapeDtypeStruct((N, D_IN), jnp.float32),
)


def _t2_body(h_ref, a0_ref, a1_ref, d0_ref, d1_ref, ws1_ref, wn1_ref,
             b1_ref, ws2_ref, wn2_ref, s_ref, p_ref):
    h = h_ref[...]
    deg = (d0_ref[0].astype(jnp.float32)
           + d1_ref[0].astype(jnp.float32))[:, :1]
    agg = (a0_ref[0] + a1_ref[0]) / jnp.maximum(deg, 1.0)
    z = (jnp.dot(h, ws1_ref[...], preferred_element_type=jnp.float32)
         + jnp.dot(agg, wn1_ref[...], preferred_element_type=jnp.float32)
         + b1_ref[...])
    n = jnp.sqrt(jnp.sum(z * z, axis=1, keepdims=True))
    z = jnp.maximum(z / (n + EPS), 0.0)
    s_ref[...] = jnp.dot(z, ws2_ref[...], preferred_element_type=jnp.float32)
    p_ref[...] = jnp.dot(z, wn2_ref[...], preferred_element_type=jnp.float32)


_t2 = pl.pallas_call(
    _t2_body,
    grid=(N // BLK,),
    in_specs=[
        pl.BlockSpec((BLK, D_IN), lambda i: (i, 0)),
        pl.BlockSpec((1, BLK, D_IN), lambda i: (0, i, 0)),
        pl.BlockSpec((1, BLK, D_IN), lambda i: (1, i, 0)),
        pl.BlockSpec((1, BLK, DP), lambda i: (0, i, 0)),
        pl.BlockSpec((1, BLK, DP), lambda i: (1, i, 0)),
        pl.BlockSpec((D_IN, D_IN), lambda i: (0, 0)),
        pl.BlockSpec((D_IN, D_IN), lambda i: (0, 0)),
        pl.BlockSpec((1, D_IN), lambda i: (0, 0)),
        pl.BlockSpec((D_IN, DP), lambda i: (0, 0)),
        pl.BlockSpec((D_IN, DP), lambda i: (0, 0)),
    ],
    out_specs=[
        pl.BlockSpec((BLK, DP), lambda i: (i, 0)),
        pl.BlockSpec((BLK, DP), lambda i: (i, 0)),
    ],
    out_shape=[
        jax.ShapeDtypeStruct((N, DP), jnp.float32),
        jax.ShapeDtypeStruct((N, DP), jnp.float32),
    ],
)


def _t3_body(s_ref, q0_ref, q1_ref, d0_ref, d1_ref, b2_ref, o_ref):
    deg = (d0_ref[0].astype(jnp.float32)
           + d1_ref[0].astype(jnp.float32))[:, :1]
    agg = (q0_ref[0] + q1_ref[0]) / jnp.maximum(deg, 1.0)
    o = s_ref[...] + agg + b2_ref[...]
    n = jnp.sqrt(jnp.sum(o * o, axis=1, keepdims=True))
    o = o / (n + EPS)
    mask = lax.broadcasted_iota(jnp.int32, o.shape, 1) < D_OUT
    o = jnp.where(mask, o, -jnp.inf)
    m = jnp.max(o, axis=1, keepdims=True)
    e = jnp.where(mask, jnp.exp(o - m), 0.0)
    lse = jnp.log(jnp.sum(e, axis=1, keepdims=True))
    o_ref[...] = (o - m - lse)[:, :D_OUT]


_t3 = pl.pallas_call(
    _t3_body,
    grid=(N // BLK,),
    in_specs=[
        pl.BlockSpec((BLK, DP), lambda i: (i, 0)),
        pl.BlockSpec((1, BLK, DP), lambda i: (0, i, 0)),
        pl.BlockSpec((1, BLK, DP), lambda i: (1, i, 0)),
        pl.BlockSpec((1, BLK, DP), lambda i: (0, i, 0)),
        pl.BlockSpec((1, BLK, DP), lambda i: (1, i, 0)),
        pl.BlockSpec((1, DP), lambda i: (0, 0)),
    ],
    out_specs=pl.BlockSpec((BLK, D_OUT), lambda i: (i, 0)),
    out_shape=jax.ShapeDtypeStruct((N, D_OUT), jnp.float32),
)


@jax.jit
def kernel(x, edge_index, W_self1, W_neigh1, b1, W_self2, W_neigh2, b2):
    src3 = edge_index[0].reshape(NW, MAIN, CH)
    dst3 = edge_index[1].reshape(NW, MAIN, CH)
    z128 = jnp.zeros((RT_LAST, D_IN), jnp.float32)
    z16 = jnp.zeros((RT_LAST, DP), jnp.float32)
    z16b = jnp.zeros((RT_LAST, DP), jnp.bfloat16)
    ones16 = jnp.ones((CH, DP), jnp.bfloat16)
    ws2p = jnp.pad(W_self2, ((0, 0), (0, DP - D_OUT)))
    wn2p = jnp.pad(W_neigh2, ((0, 0), (0, DP - D_OUT)))
    b1r = b1.reshape(1, D_IN)
    b2p = jnp.pad(b2, (0, DP - D_OUT)).reshape(1, DP)

    h = _t1(x)
    agg_parts, deg_parts = _sc_agg1(h, src3, dst3, z128, z16b, ones16)
    s, p = _t2(h, agg_parts, agg_parts, deg_parts, deg_parts,
               W_self1, W_neigh1, b1r, ws2p, wn2p)
    parts2 = _sc_agg2(p, src3, dst3, z16)
    return _t3(s, parts2, parts2, deg_parts, deg_parts, b2p)
